# Initial kernel scaffold; baseline (speedup 1.0000x reference)
import jax
import jax.numpy as jnp
from jax.experimental import pallas as pl


def kernel(x, edge_index, W1e, b1e, W2e, b2e, Wmue, bmue, Wlse, blse, W4e, b4e, W1n, b1n, Wmun, bmun, Wlsn, blsn, W5n, b5n):
    raise NotImplementedError("write your pallas kernel here")


# speedup vs baseline: 6.3348x; 6.3348x over previous
import functools

import jax
import jax.numpy as jnp
from jax import lax
from jax.experimental import pallas as pl
from jax.experimental.pallas import tpu as pltpu
from jax.experimental.pallas import tpu_sc as plsc

N = 10000
R = 10240
E = 320000
CHUNK = 128
DP = 128
NCORE = 2
NSUB = 16
NW = NCORE * NSUB
CPW = -(-E // (CHUNK * NW))
EPAD = CPW * NW * CHUNK
ROWS_PW = R // NSUB
BR = 1024


def _spmm_body(p_hbm, src_hbm, dst_hbm, zero_hbm, out_hbm, src_v, dst_v, buf, acc):
    c = lax.axis_index("c")
    s = lax.axis_index("s")
    pltpu.sync_copy(zero_hbm.at[pl.ds(s * ROWS_PW, ROWS_PW)],
                    acc.at[pl.ds(s * ROWS_PW, ROWS_PW)])
    pltpu.sync_copy(src_hbm.at[c, s], src_v)
    pltpu.sync_copy(dst_hbm.at[c, s], dst_v)
    plsc.subcore_barrier()

    def body(j, carry):
        pltpu.sync_copy(p_hbm.at[src_v.at[j]], buf)
        pltpu.sync_copy(buf, acc.at[dst_v.at[j]], add=True)
        return carry

    lax.fori_loop(0, CPW, body, 0)
    plsc.subcore_barrier()
    pltpu.sync_copy(acc.at[pl.ds(s * ROWS_PW, ROWS_PW)],
                    out_hbm.at[c, pl.ds(s * ROWS_PW, ROWS_PW)])


@functools.lru_cache(maxsize=None)
def _make_spmm():
    mesh = plsc.VectorSubcoreMesh(core_axis_name="c", subcore_axis_name="s")
    return functools.partial(
        pl.kernel,
        mesh=mesh,
        out_type=jax.ShapeDtypeStruct((NCORE, R, DP), jnp.float32),
        scratch_types=[
            pltpu.VMEM((CPW, CHUNK), jnp.int32),
            pltpu.VMEM((CPW, CHUNK), jnp.int32),
            pltpu.VMEM((CHUNK, DP), jnp.float32),
            pltpu.VMEM_SHARED((R, DP), jnp.float32),
        ],
    )(_spmm_body)


def _deg_body(ones_hbm, dst_hbm, zero_hbm, out_hbm, dst_v, buf, acc):
    c = lax.axis_index("c")
    s = lax.axis_index("s")
    pltpu.sync_copy(zero_hbm.at[pl.ds(s * ROWS_PW, ROWS_PW)],
                    acc.at[pl.ds(s * ROWS_PW, ROWS_PW)])
    pltpu.sync_copy(dst_hbm.at[c, s], dst_v)
    pltpu.sync_copy(ones_hbm, buf)
    plsc.subcore_barrier()

    def body(j, carry):
        pltpu.sync_copy(buf, acc.at[dst_v.at[j]], add=True)
        return carry

    lax.fori_loop(0, CPW, body, 0)
    plsc.subcore_barrier()
    pltpu.sync_copy(acc.at[pl.ds(s * ROWS_PW, ROWS_PW)],
                    out_hbm.at[c, pl.ds(s * ROWS_PW, ROWS_PW)])


@functools.lru_cache(maxsize=None)
def _make_deg():
    mesh = plsc.VectorSubcoreMesh(core_axis_name="c", subcore_axis_name="s")
    return functools.partial(
        pl.kernel,
        mesh=mesh,
        out_type=jax.ShapeDtypeStruct((NCORE, R, DP), jnp.float32),
        scratch_types=[
            pltpu.VMEM((CPW, CHUNK), jnp.int32),
            pltpu.VMEM((CHUNK, DP), jnp.float32),
            pltpu.VMEM_SHARED((R, DP), jnp.float32),
        ],
    )(_deg_body)


def _dinv_of(deg0_ref, deg1_ref):
    return lax.rsqrt(deg0_ref[:, 0:1] + deg1_ref[:, 0:1] + 1.0)


def _first_body(x_ref, w_ref, deg0_ref, deg1_ref, out_ref):
    dinv = _dinv_of(deg0_ref, deg1_ref)
    out_ref[...] = jnp.dot(x_ref[...], w_ref[...],
                           preferred_element_type=jnp.float32) * dinv


def _mid_body(s0_ref, s1_ref, p_ref, deg0_ref, deg1_ref, b_ref, w_ref, out_ref,
              *, act):
    dinv = _dinv_of(deg0_ref, deg1_ref)
    h = (s0_ref[...] + s1_ref[...] + p_ref[...]) * dinv + b_ref[...]
    if act == "relu":
        h = jnp.maximum(h, 0.0)
    out_ref[...] = jnp.dot(h, w_ref[...],
                           preferred_element_type=jnp.float32) * dinv


def _last_body(s0_ref, s1_ref, p_ref, deg0_ref, deg1_ref, b_ref, out_ref, *, act):
    dinv = _dinv_of(deg0_ref, deg1_ref)
    h = (s0_ref[...] + s1_ref[...] + p_ref[...]) * dinv + b_ref[...]
    if act == "relu":
        h = jnp.maximum(h, 0.0)
    else:
        h = jax.nn.sigmoid(h)
    out_ref[...] = h


def _row_spec(d):
    return pl.BlockSpec((BR, d), lambda i: (i, 0))


def _full_spec(r, c):
    return pl.BlockSpec((r, c), lambda i: (0, 0))


def _tc_first(xp, w, deg0, deg1):
    dout = w.shape[1]
    return pl.pallas_call(
        _first_body,
        grid=(R // BR,),
        in_specs=[_row_spec(xp.shape[1]), _full_spec(*w.shape),
                  _row_spec(DP), _row_spec(DP)],
        out_specs=_row_spec(dout),
        out_shape=jax.ShapeDtypeStruct((R, dout), jnp.float32),
    )(xp, w, deg0, deg1)


def _tc_mid(s, p, deg0, deg1, b, w, act):
    dprev = p.shape[1]
    dout = w.shape[1]
    return pl.pallas_call(
        functools.partial(_mid_body, act=act),
        grid=(R // BR,),
        in_specs=[_row_spec(dprev), _row_spec(dprev), _row_spec(dprev),
                  _row_spec(DP), _row_spec(DP),
                  _full_spec(1, dprev), _full_spec(*w.shape)],
        out_specs=_row_spec(dout),
        out_shape=jax.ShapeDtypeStruct((R, dout), jnp.float32),
    )(s[0], s[1], p, deg0, deg1, b, w)


def _tc_last(s, p, deg0, deg1, b, act):
    dprev = p.shape[1]
    return pl.pallas_call(
        functools.partial(_last_body, act=act),
        grid=(R // BR,),
        in_specs=[_row_spec(dprev), _row_spec(dprev), _row_spec(dprev),
                  _row_spec(DP), _row_spec(DP), _full_spec(1, dprev)],
        out_specs=_row_spec(dprev),
        out_shape=jax.ShapeDtypeStruct((R, dprev), jnp.float32),
    )(s[0], s[1], p, deg0, deg1, b)


def _pad_w(w):
    return jnp.pad(w, ((0, DP - w.shape[0]), (0, DP - w.shape[1])))


def _pad_b(b):
    return jnp.pad(b, (0, DP - b.shape[0])).reshape(1, DP)


def _chunk_idx(v):
    pad = jnp.full((EPAD - E,), N, v.dtype)
    return jnp.concatenate([v, pad]).reshape(NCORE, NSUB, CPW, CHUNK)


def kernel(x, edge_index, W1e, b1e, W2e, b2e, Wmue, bmue, Wlse, blse, W4e, b4e,
           W1n, b1n, Wmun, bmun, Wlsn, blsn, W5n, b5n):
    del Wlse, blse, Wlsn, blsn
    f32 = jnp.float32
    xp = jnp.pad(x, ((0, R - N), (0, 0)))
    srcs = _chunk_idx(edge_index[0].astype(jnp.int32))
    dsts = _chunk_idx(edge_index[1].astype(jnp.int32))
    zero = jnp.zeros((R, DP), f32)

    degS = _make_deg()(jnp.ones((CHUNK, DP), f32), dsts, zero)
    deg0, deg1 = degS[0], degS[1]

    def spmm(p):
        return _make_spmm()(p, srcs, dsts, zero)

    p = _tc_first(xp, _pad_w(W1e), deg0, deg1)
    s = spmm(p)
    p = _tc_mid(s, p, deg0, deg1, _pad_b(b1e), _pad_w(W2e), "relu")
    s = spmm(p)
    p = _tc_mid(s, p, deg0, deg1, _pad_b(b2e), _pad_w(Wmue), "relu")
    s = spmm(p)
    p = _tc_mid(s, p, deg0, deg1, _pad_b(bmue), _pad_w(W4e), "id")
    s = spmm(p)
    edges = _tc_last(s, p, deg0, deg1, _pad_b(b4e), "sigmoid")[:N, :16]

    p = _tc_first(xp, W1n, deg0, deg1)
    s = spmm(p)
    p = _tc_mid(s, p, deg0, deg1, _pad_b(b1n), Wmun, "relu")
    s = spmm(p)
    p = _tc_mid(s, p, deg0, deg1, _pad_b(bmun), W5n, "id")
    s = spmm(p)
    nodes = _tc_last(s, p, deg0, deg1, _pad_b(b5n), "relu")[:N]

    return (edges, nodes)
